# baseline (speedup 1.0000x reference)
import functools

import jax
import jax.numpy as jnp
from jax import lax
from jax.experimental import pallas as pl
from jax.experimental.pallas import tpu as pltpu
from jax.experimental.pallas import tpu_sc as plsc

_INTERPRET = False

N_NODES = 10000
N_PAD = 10240
D = 128
NC = 2
NS = 16
NW = NC * NS
CHUNK = 128
IDX_BLK = 16
ROWS_PER_SUB = N_PAD // NS


def _sc_count(n_chunks, cw):
    mesh = plsc.VectorSubcoreMesh(
        core_axis_name="c", subcore_axis_name="s", num_cores=NC, num_subcores=NS
    )

    def body(dst3, ones, zeros, c_out, cnt_sh, idx_d, ones_v):
        c = lax.axis_index("c")
        s = lax.axis_index("s")
        w = c * NS + s
        base = s * ROWS_PER_SUB
        pltpu.sync_copy(zeros, cnt_sh.at[pl.ds(base, ROWS_PER_SUB)])
        pltpu.sync_copy(ones, ones_v)
        plsc.subcore_barrier()

        def outer(o, carry):
            pltpu.sync_copy(dst3.at[w, pl.ds(o * IDX_BLK, IDX_BLK)], idx_d)
            for j in range(IDX_BLK):
                pltpu.sync_copy(ones_v, cnt_sh.at[idx_d.at[j]], add=True)
            return carry

        lax.fori_loop(0, n_chunks // IDX_BLK, outer, 0)
        plsc.subcore_barrier()
        pltpu.sync_copy(
            cnt_sh.at[pl.ds(base, ROWS_PER_SUB)],
            c_out.at[c, pl.ds(base, ROWS_PER_SUB)],
        )

    return pl.kernel(
        body,
        out_type=[jax.ShapeDtypeStruct((NC, N_PAD, cw), jnp.float32)],
        mesh=mesh,
        scratch_types=[
            pltpu.VMEM_SHARED((N_PAD, cw), jnp.float32),
            pltpu.VMEM((IDX_BLK, CHUNK), jnp.int32),
            pltpu.VMEM((CHUNK, cw), jnp.float32),
        ],
    )


def _sc_aggregate(n_outer):
    mesh = plsc.VectorSubcoreMesh(
        core_axis_name="c", subcore_axis_name="s", num_cores=NC, num_subcores=NS
    )

    def body(feat, src3, dst3, zeros, p_out, s_sh, idx_s, idx_d, rows0, rows1, sem):
        c = lax.axis_index("c")
        s = lax.axis_index("s")
        w = c * NS + s
        base = s * ROWS_PER_SUB
        pltpu.sync_copy(zeros, s_sh.at[pl.ds(base, ROWS_PER_SUB)])
        plsc.subcore_barrier()
        bufs = [rows0, rows1]

        H = CHUNK // 2

        def gather(j):
            b = bufs[j % 2]
            return (
                pltpu.async_copy(
                    feat.at[idx_s.at[j, pl.ds(0, H)]], b.at[pl.ds(0, H)], sem
                ),
                pltpu.async_copy(
                    feat.at[idx_s.at[j, pl.ds(H, H)]], b.at[pl.ds(H, H)], sem
                ),
            )

        def outer(o, carry):
            pltpu.sync_copy(src3.at[w, pl.ds(o * IDX_BLK, IDX_BLK)], idx_s)
            pltpu.sync_copy(dst3.at[w, pl.ds(o * IDX_BLK, IDX_BLK)], idx_d)
            d = gather(0)
            for j in range(IDX_BLK):
                d[0].wait()
                d[1].wait()
                if j + 1 < IDX_BLK:
                    d = gather(j + 1)
                pltpu.sync_copy(bufs[j % 2], s_sh.at[idx_d.at[j]], add=True)
            return carry

        lax.fori_loop(0, n_outer, outer, 0)
        plsc.subcore_barrier()
        pltpu.sync_copy(
            s_sh.at[pl.ds(base, ROWS_PER_SUB)],
            p_out.at[c, pl.ds(base, ROWS_PER_SUB)],
        )

    return pl.kernel(
        body,
        out_type=[jax.ShapeDtypeStruct((NC, N_PAD, D), jnp.float32)],
        mesh=mesh,
        scratch_types=[
            pltpu.VMEM_SHARED((N_PAD, D), jnp.float32),
            pltpu.VMEM((IDX_BLK, CHUNK), jnp.int32),
            pltpu.VMEM((IDX_BLK, CHUNK), jnp.int32),
            pltpu.VMEM((CHUNK, D), jnp.float32),
            pltpu.VMEM((CHUNK, D), jnp.float32),
            pltpu.SemaphoreType.DMA,
        ],
    )


def _tc_layer(relu, n_pad, cw):
    R = 256
    grid = (n_pad // R,)

    def body(p_ref, c_ref, x_ref, wl_ref, wr_ref, b_ref, o_ref):
        ssum = p_ref[0] + p_ref[1]
        cnt = c_ref[0][:, 0:1] + c_ref[1][:, 0:1]
        inv = 1.0 / jnp.maximum(cnt, 1.0)
        acc = jnp.dot(ssum * inv, wl_ref[...], preferred_element_type=jnp.float32)
        acc = acc + jnp.dot(x_ref[...], wr_ref[...], preferred_element_type=jnp.float32)
        acc = acc + b_ref[...]
        if relu:
            acc = jnp.maximum(acc, 0.0)
        o_ref[...] = acc

    return pl.pallas_call(
        body,
        grid=grid,
        in_specs=[
            pl.BlockSpec((NC, R, D), lambda i: (0, i, 0)),
            pl.BlockSpec((NC, R, cw), lambda i: (0, i, 0)),
            pl.BlockSpec((R, D), lambda i: (i, 0)),
            pl.BlockSpec((D, D), lambda i: (0, 0)),
            pl.BlockSpec((D, D), lambda i: (0, 0)),
            pl.BlockSpec((1, D), lambda i: (0, 0)),
        ],
        out_specs=pl.BlockSpec((R, D), lambda i: (i, 0)),
        out_shape=jax.ShapeDtypeStruct((n_pad, D), jnp.float32),
        interpret=_INTERPRET,
    )


def kernel(x, edge_index, Wl0, bl0, Wr0, Wl1, bl1, Wr1, Wl2, bl2, Wr2):
    e = edge_index.shape[1]
    per_w = -(-e // NW)
    n_chunks = -(-per_w // CHUNK)
    n_chunks = -(-n_chunks // IDX_BLK) * IDX_BLK
    e_pad = NW * n_chunks * CHUNK

    src = edge_index[0]
    dst = edge_index[1]
    per_w_real = e // NW
    pad_w = n_chunks * CHUNK - per_w_real
    dummy_src = (jnp.arange(NW * pad_w, dtype=jnp.int32) * 37) % N_NODES
    n_spare = N_PAD - N_NODES - 1
    dummy_dst = N_NODES + (jnp.arange(NW * pad_w, dtype=jnp.int32) % n_spare)
    src3 = jnp.concatenate(
        [src.reshape(NW, per_w_real), dummy_src.reshape(NW, pad_w)], axis=1
    ).reshape(NW, n_chunks, CHUNK)
    dst3 = jnp.concatenate(
        [dst.reshape(NW, per_w_real), dummy_dst.reshape(NW, pad_w)], axis=1
    ).reshape(NW, n_chunks, CHUNK)

    x_pad = jnp.pad(x, ((0, N_PAD - x.shape[0]), (0, 0)))
    cw = D
    ones = jnp.ones((CHUNK, cw), jnp.float32)
    zeros_cw = jnp.zeros((ROWS_PER_SUB, cw), jnp.float32)
    zeros = jnp.zeros((ROWS_PER_SUB, D), jnp.float32)

    count = _sc_count(n_chunks, cw)
    agg = _sc_aggregate(n_chunks // IDX_BLK)
    layer_relu = _tc_layer(True, N_PAD, cw)
    layer_lin = _tc_layer(False, N_PAD, cw)

    (cnt,) = count(dst3, ones, zeros_cw)
    (p0,) = agg(x_pad, src3, dst3, zeros)
    h1 = layer_relu(p0, cnt, x_pad, Wl0.T, Wr0.T, bl0.reshape(1, D))
    (p1,) = agg(h1, src3, dst3, zeros)
    h2 = layer_relu(p1, cnt, h1, Wl1.T, Wr1.T, bl1.reshape(1, D))
    (p2,) = agg(h2, src3, dst3, zeros)
    out = layer_lin(p2, cnt, h2, Wl2.T, Wr2.T, bl2.reshape(1, D))
    return out[:N_NODES]


# speedup vs baseline: 1.0136x; 1.0136x over previous
import functools

import jax
import jax.numpy as jnp
from jax import lax
from jax.experimental import pallas as pl
from jax.experimental.pallas import tpu as pltpu
from jax.experimental.pallas import tpu_sc as plsc

_INTERPRET = False

N_NODES = 10000
N_PAD = 10240
D = 128
NC = 2
NS = 16
NW = NC * NS
CHUNK = 128
IDX_BLK = 16
ROWS_PER_SUB = N_PAD // NS


def _sc_count(n_chunks, cw):
    mesh = plsc.VectorSubcoreMesh(
        core_axis_name="c", subcore_axis_name="s", num_cores=NC, num_subcores=NS
    )

    def body(dst3, ones, zeros, c_out, cnt_sh, idx_d, ones_v):
        c = lax.axis_index("c")
        s = lax.axis_index("s")
        w = c * NS + s
        base = s * ROWS_PER_SUB
        pltpu.sync_copy(zeros, cnt_sh.at[pl.ds(base, ROWS_PER_SUB)])
        pltpu.sync_copy(ones, ones_v)
        plsc.subcore_barrier()

        def outer(o, carry):
            pltpu.sync_copy(dst3.at[w, pl.ds(o * IDX_BLK, IDX_BLK)], idx_d)
            for j in range(IDX_BLK):
                pltpu.sync_copy(ones_v, cnt_sh.at[idx_d.at[j]], add=True)
            return carry

        lax.fori_loop(0, n_chunks // IDX_BLK, outer, 0)
        plsc.subcore_barrier()
        pltpu.sync_copy(
            cnt_sh.at[pl.ds(base, ROWS_PER_SUB)],
            c_out.at[c, pl.ds(base, ROWS_PER_SUB)],
        )

    return pl.kernel(
        body,
        out_type=[jax.ShapeDtypeStruct((NC, N_PAD, cw), jnp.float32)],
        mesh=mesh,
        scratch_types=[
            pltpu.VMEM_SHARED((N_PAD, cw), jnp.float32),
            pltpu.VMEM((IDX_BLK, CHUNK), jnp.int32),
            pltpu.VMEM((CHUNK, cw), jnp.float32),
        ],
    )


def _sc_aggregate(n_outer, with_cnt=False):
    mesh = plsc.VectorSubcoreMesh(
        core_axis_name="c", subcore_axis_name="s", num_cores=NC, num_subcores=NS
    )

    out_type = [jax.ShapeDtypeStruct((NC, N_PAD, D), jnp.float32)]
    if with_cnt:
        out_type.append(jax.ShapeDtypeStruct((NC, N_PAD, D), jnp.float32))

    def body(feat, src3, dst3, zeros, ones, *refs):
        if with_cnt:
            p_out, c_out, s_sh, idx_s, idx_d, rows0, rows1, sem = refs
        else:
            p_out, s_sh, idx_s, idx_d, rows0, rows1, sem = refs
        c = lax.axis_index("c")
        s = lax.axis_index("s")
        w = c * NS + s
        base = s * ROWS_PER_SUB
        pltpu.sync_copy(zeros, s_sh.at[pl.ds(base, ROWS_PER_SUB)])
        bufs = [rows0, rows1]

        if with_cnt:
            pltpu.sync_copy(ones, rows0)
            plsc.subcore_barrier()

            def cnt_outer(o, carry):
                pltpu.sync_copy(dst3.at[w, pl.ds(o * IDX_BLK, IDX_BLK)], idx_d)
                for j in range(IDX_BLK):
                    pltpu.sync_copy(rows0, s_sh.at[idx_d.at[j]], add=True)
                return carry

            lax.fori_loop(0, n_outer, cnt_outer, 0)
            plsc.subcore_barrier()
            pltpu.sync_copy(
                s_sh.at[pl.ds(base, ROWS_PER_SUB)],
                c_out.at[c, pl.ds(base, ROWS_PER_SUB)],
            )
            pltpu.sync_copy(zeros, s_sh.at[pl.ds(base, ROWS_PER_SUB)])
        plsc.subcore_barrier()

        def outer(o, carry):
            pltpu.sync_copy(src3.at[w, pl.ds(o * IDX_BLK, IDX_BLK)], idx_s)
            pltpu.sync_copy(dst3.at[w, pl.ds(o * IDX_BLK, IDX_BLK)], idx_d)
            d = pltpu.async_copy(feat.at[idx_s.at[0]], bufs[0], sem)
            for j in range(IDX_BLK):
                d.wait()
                if j + 1 < IDX_BLK:
                    d = pltpu.async_copy(
                        feat.at[idx_s.at[j + 1]], bufs[(j + 1) % 2], sem
                    )
                pltpu.sync_copy(bufs[j % 2], s_sh.at[idx_d.at[j]], add=True)
            return carry

        lax.fori_loop(0, n_outer, outer, 0)
        plsc.subcore_barrier()
        pltpu.sync_copy(
            s_sh.at[pl.ds(base, ROWS_PER_SUB)],
            p_out.at[c, pl.ds(base, ROWS_PER_SUB)],
        )

    return pl.kernel(
        body,
        out_type=out_type,
        mesh=mesh,
        scratch_types=[
            pltpu.VMEM_SHARED((N_PAD, D), jnp.float32),
            pltpu.VMEM((IDX_BLK, CHUNK), jnp.int32),
            pltpu.VMEM((IDX_BLK, CHUNK), jnp.int32),
            pltpu.VMEM((CHUNK, D), jnp.float32),
            pltpu.VMEM((CHUNK, D), jnp.float32),
            pltpu.SemaphoreType.DMA,
        ],
    )


def _tc_layer(relu, n_pad, cw):
    R = 256
    grid = (n_pad // R,)

    def body(p_ref, c_ref, x_ref, wl_ref, wr_ref, b_ref, o_ref):
        ssum = p_ref[0] + p_ref[1]
        cnt = c_ref[0][:, 0:1] + c_ref[1][:, 0:1]
        inv = 1.0 / jnp.maximum(cnt, 1.0)
        acc = jnp.dot(ssum * inv, wl_ref[...], preferred_element_type=jnp.float32)
        acc = acc + jnp.dot(x_ref[...], wr_ref[...], preferred_element_type=jnp.float32)
        acc = acc + b_ref[...]
        if relu:
            acc = jnp.maximum(acc, 0.0)
        o_ref[...] = acc

    return pl.pallas_call(
        body,
        grid=grid,
        in_specs=[
            pl.BlockSpec((NC, R, D), lambda i: (0, i, 0)),
            pl.BlockSpec((NC, R, cw), lambda i: (0, i, 0)),
            pl.BlockSpec((R, D), lambda i: (i, 0)),
            pl.BlockSpec((D, D), lambda i: (0, 0)),
            pl.BlockSpec((D, D), lambda i: (0, 0)),
            pl.BlockSpec((1, D), lambda i: (0, 0)),
        ],
        out_specs=pl.BlockSpec((R, D), lambda i: (i, 0)),
        out_shape=jax.ShapeDtypeStruct((n_pad, D), jnp.float32),
        interpret=_INTERPRET,
    )


def kernel(x, edge_index, Wl0, bl0, Wr0, Wl1, bl1, Wr1, Wl2, bl2, Wr2):
    e = edge_index.shape[1]
    per_w = -(-e // NW)
    n_chunks = -(-per_w // CHUNK)
    n_chunks = -(-n_chunks // IDX_BLK) * IDX_BLK
    e_pad = NW * n_chunks * CHUNK

    src = edge_index[0]
    dst = edge_index[1]
    per_w_real = e // NW
    pad_w = n_chunks * CHUNK - per_w_real
    dummy_src = (jnp.arange(NW * pad_w, dtype=jnp.int32) * 37) % N_NODES
    n_spare = N_PAD - N_NODES - 1
    dummy_dst = N_NODES + (jnp.arange(NW * pad_w, dtype=jnp.int32) % n_spare)
    src3 = jnp.concatenate(
        [src.reshape(NW, per_w_real), dummy_src.reshape(NW, pad_w)], axis=1
    ).reshape(NW, n_chunks, CHUNK)
    dst3 = jnp.concatenate(
        [dst.reshape(NW, per_w_real), dummy_dst.reshape(NW, pad_w)], axis=1
    ).reshape(NW, n_chunks, CHUNK)

    x_pad = jnp.pad(x, ((0, N_PAD - x.shape[0]), (0, 0)))
    ones = jnp.ones((CHUNK, D), jnp.float32)
    zeros = jnp.zeros((ROWS_PER_SUB, D), jnp.float32)

    agg0 = _sc_aggregate(n_chunks // IDX_BLK, with_cnt=True)
    agg = _sc_aggregate(n_chunks // IDX_BLK)
    layer_relu = _tc_layer(True, N_PAD, D)
    layer_lin = _tc_layer(False, N_PAD, D)

    p0, cnt = agg0(x_pad, src3, dst3, zeros, ones)
    h1 = layer_relu(p0, cnt, x_pad, Wl0.T, Wr0.T, bl0.reshape(1, D))
    (p1,) = agg(h1, src3, dst3, zeros, ones)
    h2 = layer_relu(p1, cnt, h1, Wl1.T, Wr1.T, bl1.reshape(1, D))
    (p2,) = agg(h2, src3, dst3, zeros, ones)
    out = layer_lin(p2, cnt, h2, Wl2.T, Wr2.T, bl2.reshape(1, D))
    return out[:N_NODES]
